# baseline (speedup 1.0000x reference)
import functools
import jax
import jax.numpy as jnp
from jax import lax
from jax.experimental import pallas as pl
from jax.experimental.pallas import tpu as pltpu
from jax.experimental.pallas import tpu_sc as plsc

B, S, V = 256, 100, 128
ROW = V * 3
NW = 32
ITEMS_PER_W = B // NW


def _sc_body(x_hbm, out_hbm, buf, out_row):
    wid = lax.axis_index("s") * 2 + lax.axis_index("c")
    lane = lax.iota(jnp.int32, 16)

    def per_item(i, _):
        b = wid * ITEMS_PER_W + i
        pltpu.sync_copy(x_hbm.at[b], buf)

        for g in range(V // 16):
            base3 = g * 48 + lane * 3

            def scan_s(s, m):
                cls = plsc.load_gather(buf, [s * ROW + base3])
                cand = jnp.where(cls > 0.0, s + 1, 0)
                return jnp.maximum(m, cand)

            m = lax.fori_loop(0, S, scan_s, jnp.zeros((16,), jnp.int32))

            found = m > 0
            base = jnp.where(found, (m - 1) * ROW, 0) + base3
            x = plsc.load_gather(buf, [base + 1])
            y = plsc.load_gather(buf, [base + 2])
            x = jnp.where(found, x, 0.0)
            y = jnp.where(found, y, 0.0)
            plsc.store_scatter(out_row, [base3], jnp.ones((16,), jnp.float32))
            plsc.store_scatter(out_row, [base3 + 1], x)
            plsc.store_scatter(out_row, [base3 + 2], y)

        pltpu.sync_copy(out_row, out_hbm.at[b])
        return 0

    lax.fori_loop(0, ITEMS_PER_W, per_item, 0)


def kernel(batch):
    x = batch.reshape(B, S * ROW)
    mesh = plsc.VectorSubcoreMesh(core_axis_name="c", subcore_axis_name="s")
    k = pl.kernel(
        _sc_body,
        out_type=jax.ShapeDtypeStruct((B, ROW), jnp.float32),
        mesh=mesh,
        scratch_types=[
            pltpu.VMEM((S * ROW,), jnp.float32),
            pltpu.VMEM((ROW,), jnp.float32),
        ],
        compiler_params=pltpu.CompilerParams(needs_layout_passes=False),
    )
    out = k(x)
    return out.reshape(B, V, 3)


# speedup vs baseline: 1.2008x; 1.2008x over previous
import jax
import jax.numpy as jnp
from jax import lax
from jax.experimental import pallas as pl
from jax.experimental.pallas import tpu as pltpu
from jax.experimental.pallas import tpu_sc as plsc

B, S, V = 256, 100, 128
ROW = V * 3
NW = 32
ITEMS_PER_W = B // NW
CH = 10
NCH = S // CH
NG = V // 16


def _sc_body(x_hbm, out_hbm, buf, out_row, m_ref):
    wid = lax.axis_index("s") * 2 + lax.axis_index("c")
    lane = lax.iota(jnp.int32, 16)
    zero16 = jnp.zeros((16,), jnp.int32)

    def per_item(i, _):
        b = wid * ITEMS_PER_W + i
        for g in range(NG):
            m_ref[pl.ds(g * 16, 16)] = zero16

        def chunk_cond(carry):
            c, gmin = carry
            return jnp.logical_and(c < NCH, gmin == 0)

        def chunk_body(carry):
            c, _ = carry
            s_top = (S - 1) - CH * c
            off = (s_top - (CH - 1)) * ROW
            pltpu.sync_copy(
                x_hbm.at[b, pl.ds(off, CH * ROW)], buf.at[pl.ds(off, CH * ROW)]
            )
            ms = []
            for g in range(NG):
                base3 = g * 48 + lane * 3
                cands = []
                for j in range(CH):
                    s = s_top - j
                    cls = plsc.load_gather(buf, [s * ROW + base3])
                    cands.append(jnp.where(cls > 0.0, s + 1, 0))
                while len(cands) > 1:
                    cands = [
                        jnp.maximum(cands[k], cands[k + 1])
                        for k in range(0, len(cands) - 1, 2)
                    ] + ([cands[-1]] if len(cands) % 2 else [])
                m = m_ref[pl.ds(g * 16, 16)]
                m = jnp.where(m > 0, m, cands[0])
                m_ref[pl.ds(g * 16, 16)] = m
                ms.append(m)
            while len(ms) > 1:
                ms = [
                    jnp.minimum(ms[k], ms[k + 1]) for k in range(0, len(ms) - 1, 2)
                ] + ([ms[-1]] if len(ms) % 2 else [])
            return c + 1, jnp.min(ms[0])

        lax.while_loop(chunk_cond, chunk_body, (0, 0))

        for g in range(NG):
            base3 = g * 48 + lane * 3
            m = m_ref[pl.ds(g * 16, 16)]
            found = m > 0
            base = jnp.where(found, (m - 1) * ROW, 0) + base3
            x = plsc.load_gather(buf, [base + 1])
            y = plsc.load_gather(buf, [base + 2])
            x = jnp.where(found, x, 0.0)
            y = jnp.where(found, y, 0.0)
            plsc.store_scatter(out_row, [base3], jnp.ones((16,), jnp.float32))
            plsc.store_scatter(out_row, [base3 + 1], x)
            plsc.store_scatter(out_row, [base3 + 2], y)

        pltpu.sync_copy(out_row, out_hbm.at[b])
        return 0

    lax.fori_loop(0, ITEMS_PER_W, per_item, 0)


def kernel(batch):
    x = batch.reshape(B, S * ROW)
    mesh = plsc.VectorSubcoreMesh(core_axis_name="c", subcore_axis_name="s")
    k = pl.kernel(
        _sc_body,
        out_type=jax.ShapeDtypeStruct((B, ROW), jnp.float32),
        mesh=mesh,
        scratch_types=[
            pltpu.VMEM((S * ROW,), jnp.float32),
            pltpu.VMEM((ROW,), jnp.float32),
            pltpu.VMEM((V,), jnp.int32),
        ],
        compiler_params=pltpu.CompilerParams(needs_layout_passes=False),
    )
    out = k(x)
    return out.reshape(B, V, 3)
